# baseline (speedup 1.0000x reference)
import functools

import jax
import jax.numpy as jnp
from jax import lax
from jax.experimental import pallas as pl
from jax.experimental.pallas import tpu as pltpu
from jax.experimental.pallas import tpu_sc as plsc

D_MODEL = 64
_INFO = plsc.get_sparse_core_info()
_NC, _NS = _INFO.num_cores, _INFO.num_subcores
_NW = _NC * _NS

_CHUNK = 512
_SUB = 128
_NSUB = _CHUNK // _SUB


def _emb_call(total_rows):
    n_per_w = total_rows // _NW
    n_chunks = n_per_w // _CHUNK
    mesh = plsc.VectorSubcoreMesh(core_axis_name="c", subcore_axis_name="s")

    @functools.partial(
        pl.kernel,
        out_type=jax.ShapeDtypeStruct((total_rows, D_MODEL), jnp.float32),
        mesh=mesh,
        scratch_types=[
            pltpu.VMEM((_NSUB, _SUB), jnp.int32),
            pltpu.VMEM((_CHUNK, D_MODEL), jnp.float32),
            pltpu.SemaphoreType.DMA,
        ],
        compiler_params=pltpu.CompilerParams(use_tc_tiling_on_sc=False),
    )
    def emb(w_hbm, x_hbm, out_hbm, idx_v, rows_v, sem):
        wid = lax.axis_index("s") * _NC + lax.axis_index("c")
        base = wid * (n_per_w // _SUB)

        def step(i, carry):
            g = base + i * _NSUB
            pltpu.sync_copy(x_hbm.at[pl.ds(g, _NSUB)], idx_v)
            for j in range(_NSUB):
                pltpu.async_copy(
                    w_hbm.at[idx_v.at[j]],
                    rows_v.at[pl.ds(j * _SUB, _SUB)],
                    sem,
                )
            for j in range(_NSUB):
                pltpu.make_async_copy(
                    w_hbm.at[idx_v.at[j]],
                    rows_v.at[pl.ds(j * _SUB, _SUB)],
                    sem,
                ).wait()
            pltpu.sync_copy(
                rows_v, out_hbm.at[pl.ds(g * _SUB, _CHUNK)]
            )
            return carry

        lax.fori_loop(0, n_chunks, step, 0)

    return emb


def kernel(x, weight):
    b, s = x.shape
    total = b * s
    x2 = x.reshape(total // _SUB, _SUB).astype(jnp.int32)
    out = _emb_call(total)(weight, x2)
    return out.reshape(b, s, D_MODEL)


# speedup vs baseline: 1.0417x; 1.0417x over previous
import functools

import jax
import jax.numpy as jnp
from jax import lax
from jax.experimental import pallas as pl
from jax.experimental.pallas import tpu as pltpu
from jax.experimental.pallas import tpu_sc as plsc

D_MODEL = 64
_INFO = plsc.get_sparse_core_info()
_NC, _NS = _INFO.num_cores, _INFO.num_subcores
_NW = _NC * _NS

_CHUNK = 512
_SUB = 128
_NSUB = _CHUNK // _SUB


def _emb_call(total_rows):
    n_per_w = total_rows // _NW
    n_chunks = n_per_w // _CHUNK
    n_pairs = n_chunks // 2
    mesh = plsc.VectorSubcoreMesh(core_axis_name="c", subcore_axis_name="s")

    @functools.partial(
        pl.kernel,
        out_type=jax.ShapeDtypeStruct((total_rows, D_MODEL), jnp.float32),
        mesh=mesh,
        scratch_types=[
            pltpu.VMEM((_NSUB, _SUB), jnp.int32),
            pltpu.VMEM((_NSUB, _SUB), jnp.int32),
            pltpu.VMEM((_CHUNK, D_MODEL), jnp.float32),
            pltpu.VMEM((_CHUNK, D_MODEL), jnp.float32),
            pltpu.SemaphoreType.DMA,
            pltpu.SemaphoreType.DMA,
            pltpu.SemaphoreType.DMA,
            pltpu.SemaphoreType.DMA,
            pltpu.SemaphoreType.DMA,
            pltpu.SemaphoreType.DMA,
        ],
        compiler_params=pltpu.CompilerParams(use_tc_tiling_on_sc=False),
    )
    def emb(w_hbm, x_hbm, out_hbm, idx0, idx1, rows0, rows1,
            isem0, isem1, gsem0, gsem1, osem0, osem1):
        wid = lax.axis_index("s") * _NC + lax.axis_index("c")
        base = wid * (n_per_w // _SUB)
        bufs = ((idx0, rows0, isem0, gsem0, osem0),
                (idx1, rows1, isem1, gsem1, osem1))

        for b in (0, 1):
            idxv, _, isem, _, _ = bufs[b]
            pltpu.async_copy(x_hbm.at[pl.ds(base + b * _NSUB, _NSUB)],
                             idxv, isem)

        def pair(p, carry):
            for b in (0, 1):
                idxv, rowsv, isem, gsem, osem = bufs[b]
                i = 2 * p + b
                g = base + i * _NSUB

                @pl.when(p > 0)
                def _():
                    pltpu.make_async_copy(
                        rowsv, out_hbm.at[pl.ds((g - 2 * _NSUB) * _SUB,
                                                _CHUNK)], osem).wait()

                pltpu.make_async_copy(
                    x_hbm.at[pl.ds(g, _NSUB)], idxv, isem).wait()

                for j in range(_NSUB):
                    pltpu.async_copy(w_hbm.at[idxv.at[j]],
                                     rowsv.at[pl.ds(j * _SUB, _SUB)], gsem)
                for j in range(_NSUB):
                    pltpu.make_async_copy(
                        w_hbm.at[idxv.at[j]],
                        rowsv.at[pl.ds(j * _SUB, _SUB)], gsem).wait()

                g_next = base + lax.rem(i + 2, n_chunks) * _NSUB
                pltpu.async_copy(x_hbm.at[pl.ds(g_next, _NSUB)], idxv, isem)

                pltpu.async_copy(rowsv, out_hbm.at[pl.ds(g * _SUB, _CHUNK)],
                                 osem)
            return carry

        lax.fori_loop(0, n_pairs, pair, 0)

        for b in (0, 1):
            idxv, rowsv, isem, _, osem = bufs[b]
            i = n_chunks - 2 + b
            g = base + i * _NSUB
            pltpu.make_async_copy(
                rowsv, out_hbm.at[pl.ds(g * _SUB, _CHUNK)], osem).wait()
            pltpu.make_async_copy(
                x_hbm.at[pl.ds(base + b * _NSUB, _NSUB)], idxv, isem).wait()

    return emb


def kernel(x, weight):
    b, s = x.shape
    total = b * s
    x2 = x.reshape(total // _SUB, _SUB).astype(jnp.int32)
    out = _emb_call(total)(weight, x2)
    return out.reshape(b, s, D_MODEL)


# speedup vs baseline: 1.3912x; 1.3355x over previous
import functools

import jax
import jax.numpy as jnp
from jax import lax
from jax.experimental import pallas as pl
from jax.experimental.pallas import tpu as pltpu
from jax.experimental.pallas import tpu_sc as plsc

D_MODEL = 64
_INFO = plsc.get_sparse_core_info()
_NC, _NS = _INFO.num_cores, _INFO.num_subcores
_NW = _NC * _NS

_CHUNK = 512
_SUB = 128
_NSUB = _CHUNK // _SUB


def _emb_call(total_rows):
    n_per_w = total_rows // _NW
    n_chunks = n_per_w // _CHUNK
    n_pairs = n_chunks // 2
    mesh = plsc.VectorSubcoreMesh(core_axis_name="c", subcore_axis_name="s")

    @functools.partial(
        pl.kernel,
        out_type=jax.ShapeDtypeStruct((total_rows, 128), jnp.float32),
        mesh=mesh,
        scratch_types=[
            pltpu.VMEM((_NSUB, _SUB), jnp.int32),
            pltpu.VMEM((_NSUB, _SUB), jnp.int32),
            pltpu.VMEM((_CHUNK, D_MODEL), jnp.float32),
            pltpu.VMEM((_CHUNK, D_MODEL), jnp.float32),
            pltpu.SemaphoreType.DMA,
            pltpu.SemaphoreType.DMA,
            pltpu.SemaphoreType.DMA,
            pltpu.SemaphoreType.DMA,
            pltpu.SemaphoreType.DMA,
            pltpu.SemaphoreType.DMA,
        ],
        compiler_params=pltpu.CompilerParams(use_tc_tiling_on_sc=False),
    )
    def emb(w_hbm, x_hbm, out_hbm, idx0, idx1, rows0, rows1,
            isem0, isem1, gsem0, gsem1, osem0, osem1):
        wid = lax.axis_index("s") * _NC + lax.axis_index("c")
        base = wid * (n_per_w // _SUB)
        bufs = ((idx0, rows0, isem0, gsem0, osem0),
                (idx1, rows1, isem1, gsem1, osem1))

        for b in (0, 1):
            idxv, _, isem, _, _ = bufs[b]
            pltpu.async_copy(x_hbm.at[pl.ds(base + b * _NSUB, _NSUB)],
                             idxv, isem)

        def pair(p, carry):
            for b in (0, 1):
                idxv, rowsv, isem, gsem, osem = bufs[b]
                i = 2 * p + b
                g = base + i * _NSUB

                @pl.when(p > 0)
                def _():
                    pltpu.make_async_copy(
                        rowsv, out_hbm.at[pl.ds((g - 2 * _NSUB) * _SUB,
                                                _CHUNK),
                                          pl.ds(0, D_MODEL)], osem).wait()

                pltpu.make_async_copy(
                    x_hbm.at[pl.ds(g, _NSUB)], idxv, isem).wait()

                for j in range(_NSUB):
                    pltpu.async_copy(w_hbm.at[idxv.at[j]],
                                     rowsv.at[pl.ds(j * _SUB, _SUB)], gsem)
                for j in range(_NSUB):
                    pltpu.make_async_copy(
                        w_hbm.at[idxv.at[j]],
                        rowsv.at[pl.ds(j * _SUB, _SUB)], gsem).wait()

                g_next = base + lax.rem(i + 2, n_chunks) * _NSUB
                pltpu.async_copy(x_hbm.at[pl.ds(g_next, _NSUB)], idxv, isem)

                pltpu.async_copy(rowsv,
                                 out_hbm.at[pl.ds(g * _SUB, _CHUNK),
                                            pl.ds(0, D_MODEL)], osem)
            return carry

        lax.fori_loop(0, n_pairs, pair, 0)

        for b in (0, 1):
            idxv, rowsv, isem, _, osem = bufs[b]
            i = n_chunks - 2 + b
            g = base + i * _NSUB
            pltpu.make_async_copy(
                rowsv, out_hbm.at[pl.ds(g * _SUB, _CHUNK),
                                  pl.ds(0, D_MODEL)], osem).wait()
            pltpu.make_async_copy(
                x_hbm.at[pl.ds(base + b * _NSUB, _NSUB)], idxv, isem).wait()

    return emb


def kernel(x, weight):
    b, s = x.shape
    total = b * s
    x2 = x.reshape(total // _SUB, _SUB).astype(jnp.int32)
    out = _emb_call(total)(weight, x2)
    return out.reshape(b, s, 128)[:, :, :D_MODEL]
